# baseline (speedup 1.0000x reference)
import functools

import jax
import jax.numpy as jnp
from jax import lax
from jax.experimental import pallas as pl
from jax.experimental.pallas import tpu as pltpu
from jax.experimental.pallas import tpu_sc as plsc

B = 256
G = 8192
K = int(0.6 * G)
NC, NS, L = 2, 16, 16
NW = NC * NS
ROWS = B // NW
NCHUNK = G // L
NBINS = 256
HIST = L * NBINS

_mesh = plsc.VectorSubcoreMesh(
    core_axis_name="c", subcore_axis_name="s", num_cores=NC, num_subcores=NS
)


@functools.partial(
    pl.kernel,
    out_type=jax.ShapeDtypeStruct((B * G,), jnp.int32),
    mesh=_mesh,
    scratch_types=[
        pltpu.VMEM((G * 3,), jnp.float32),
        pltpu.VMEM((G,), jnp.int32),
        pltpu.VMEM((G,), jnp.int32),
        pltpu.VMEM((HIST,), jnp.int32),
        pltpu.VMEM((L,), jnp.int32),
    ],
    compiler_params=pltpu.CompilerParams(
        use_tc_tiling_on_sc=False, needs_layout_passes=False
    ),
)
def _mask_kernel(center_hbm, aidx_hbm, out_hbm, pts, bits, mask, hist, ivv):
    wid = lax.axis_index("s") * NC + lax.axis_index("c")
    zeros = jnp.zeros((L,), jnp.int32)
    ones = jnp.ones((L,), jnp.int32)
    lane = lax.iota(jnp.int32, L)
    lane_off = lane * NBINS
    stride3 = lane * 3

    def _clr(i, c):
        hist[pl.ds(i * L, L)] = zeros
        return c

    lax.fori_loop(0, HIST // L, _clr, 0, unroll=8)

    def scan_hist(r0):
        def grp(j, carry):
            tot, b_acc, below = carry
            s = zeros
            for l in range(L):
                off = j * L
                v = hist[pl.ds(off + l * NBINS, L)]
                hist[pl.ds(off + l * NBINS, L)] = zeros
                s = s + v
            cum = plsc.cumsum(s) + tot
            le = cum <= r0
            b_acc = b_acc + jnp.sum(jnp.where(le, 1, 0))
            below = below + jnp.sum(jnp.where(le, s, 0))
            tot = tot + jnp.sum(s)
            return (tot, b_acc, below)

        _, b, below = lax.fori_loop(0, NBINS // L, grp, (0, 0, 0))
        return b, r0 - below

    def row_body(i, c):
        row = wid * ROWS + i
        pltpu.sync_copy(center_hbm.at[pl.ds(row * (G * 3), G * 3)], pts)
        pltpu.sync_copy(aidx_hbm.at[pl.ds(row * L, L)], ivv)
        iv = ivv[...]
        ax = plsc.load_gather(pts, [iv])
        ay = plsc.load_gather(pts, [iv + 1])
        az = plsc.load_gather(pts, [iv + 2])

        def pass_a(cc, c2):
            i0 = cc * (3 * L) + stride3
            x = plsc.load_gather(pts, [i0])
            y = plsc.load_gather(pts, [i0 + 1])
            z = plsc.load_gather(pts, [i0 + 2])
            dx = x - ax
            dy = y - ay
            dz = z - az
            d2 = dx * dx + dy * dy + dz * dz
            bv = plsc.bitcast(d2, jnp.int32)
            bits[pl.ds(cc * L, L)] = bv
            bkt = lax.shift_right_logical(bv, 23)
            plsc.addupdate_scatter(hist, [lane_off + bkt], ones)
            return c2

        lax.fori_loop(0, NCHUNK, pass_a, 0, unroll=8)
        b1, r0 = scan_hist(K - 1)

        def make_pass(hi_shift, lo_shift, width_mask, prefix):
            def p(cc, c2):
                bv = bits[pl.ds(cc * L, L)]
                m = lax.shift_right_logical(bv, hi_shift) == prefix
                bkt = jnp.bitwise_and(
                    lax.shift_right_logical(bv, lo_shift), width_mask
                )
                plsc.addupdate_scatter(hist, [lane_off + bkt], ones, mask=m)
                return c2

            return p

        lax.fori_loop(0, NCHUNK, make_pass(23, 15, 0xFF, b1), 0, unroll=8)
        b2, r0 = scan_hist(r0)
        p2 = b1 * 256 + b2
        lax.fori_loop(0, NCHUNK, make_pass(15, 7, 0xFF, p2), 0, unroll=8)
        b3, r0 = scan_hist(r0)
        p3 = p2 * 256 + b3
        lax.fori_loop(0, NCHUNK, make_pass(7, 0, 0x7F, p3), 0, unroll=8)
        b4, r0 = scan_hist(r0)
        t = p3 * 128 + b4
        need = r0 + 1

        def pass_e(cc, cnt):
            bv = bits[pl.ds(cc * L, L)]
            lt = bv < t
            eq = bv == t
            eqi = jnp.where(eq, 1, 0)
            pc = plsc.cumsum(eqi) + cnt
            sel = jnp.logical_and(eq, pc <= need)
            mask[pl.ds(cc * L, L)] = jnp.where(jnp.logical_or(lt, sel), 1, 0)
            return cnt + jnp.sum(eqi)

        lax.fori_loop(0, NCHUNK, pass_e, 0, unroll=8)
        pltpu.sync_copy(mask, out_hbm.at[pl.ds(row * G, G)])
        return c

    lax.fori_loop(0, ROWS, row_body, 0)


def kernel(center):
    b, g, _ = center.shape
    idx_key = jax.random.key(42)
    rand_index = jax.random.randint(idx_key, (b,), 0, g)
    aidx = jnp.broadcast_to(
        (rand_index.astype(jnp.int32) * 3)[:, None], (b, L)
    )
    aidx = jnp.asarray(aidx, jnp.int32).reshape(b * L)
    flat = center.reshape(b * g * 3)
    out = _mask_kernel(flat, aidx)
    return out.reshape(b, g).astype(jnp.bool_)


# speedup vs baseline: 20.3142x; 20.3142x over previous
import functools

import jax
import jax.numpy as jnp
from jax import lax
from jax.experimental import pallas as pl
from jax.experimental.pallas import tpu as pltpu
from jax.experimental.pallas import tpu_sc as plsc

B = 256
G = 8192
K = int(0.6 * G)
NC, NS, L = 2, 16, 16
NW = NC * NS
ROWS = B // NW
NCHUNK = G // L
NBINS = 256
HIST = L * NBINS

_mesh = plsc.VectorSubcoreMesh(
    core_axis_name="c", subcore_axis_name="s", num_cores=NC, num_subcores=NS
)


@functools.partial(
    pl.kernel,
    out_type=jax.ShapeDtypeStruct((B, G), jnp.int32),
    mesh=_mesh,
    scratch_types=[
        pltpu.VMEM((G * 3,), jnp.float32),
        pltpu.VMEM((G,), jnp.int32),
        pltpu.VMEM((G,), jnp.int32),
        pltpu.VMEM((HIST,), jnp.int32),
        pltpu.VMEM((ROWS * L,), jnp.int32),
    ],
    compiler_params=pltpu.CompilerParams(
        use_tc_tiling_on_sc=False, needs_layout_passes=False
    ),
)
def _mask_kernel(center_hbm, aidx_hbm, out_hbm, pts, bits, mask, hist, ivv):
    wid = lax.axis_index("s") * NC + lax.axis_index("c")
    zeros = jnp.zeros((L,), jnp.int32)
    ones = jnp.ones((L,), jnp.int32)
    lane = lax.iota(jnp.int32, L)
    lane_off = lane * NBINS
    stride3 = lane * 3

    pltpu.sync_copy(aidx_hbm.at[pl.ds(wid * (ROWS * L), ROWS * L)], ivv)

    def _clr(i, c):
        hist[pl.ds(i * L, L)] = zeros
        return c

    lax.fori_loop(0, HIST // L, _clr, 0, unroll=8)

    def scan_hist(r0):
        def grp(j, carry):
            tot, b_acc, below, hb = carry
            s = zeros
            for l in range(L):
                off = j * L
                v = hist[pl.ds(off + l * NBINS, L)]
                hist[pl.ds(off + l * NBINS, L)] = zeros
                s = s + v
            cum = plsc.cumsum(s) + tot
            le = cum <= r0
            cross = jnp.logical_and(cum - s <= r0, jnp.logical_not(le))
            b_acc = b_acc + jnp.sum(jnp.where(le, 1, 0))
            below = below + jnp.sum(jnp.where(le, s, 0))
            hb = hb + jnp.sum(jnp.where(cross, s, 0))
            tot = tot + jnp.sum(s)
            return (tot, b_acc, below, hb)

        _, b, below, hb = lax.fori_loop(0, NBINS // L, grp, (0, 0, 0, 0))
        return b, r0 - below, hb

    def row_body(i, c):
        row = wid * ROWS + i
        pltpu.sync_copy(center_hbm.at[row], pts)
        iv = ivv[pl.ds(i * L, L)]
        ax = plsc.load_gather(pts, [iv])
        ay = plsc.load_gather(pts, [iv + 1])
        az = plsc.load_gather(pts, [iv + 2])

        def pass_a(cc, c2):
            i0 = cc * (3 * L) + stride3
            x = plsc.load_gather(pts, [i0])
            y = plsc.load_gather(pts, [i0 + 1])
            z = plsc.load_gather(pts, [i0 + 2])
            dx = x - ax
            dy = y - ay
            dz = z - az
            d2 = dx * dx + dy * dy + dz * dz
            bv = plsc.bitcast(d2, jnp.int32)
            bits[pl.ds(cc * L, L)] = bv
            bkt = lax.shift_right_logical(bv, 23)
            plsc.addupdate_scatter(hist, [lane_off + bkt], ones)
            return c2

        lax.fori_loop(0, NCHUNK, pass_a, 0, unroll=8)
        b1, r0, _ = scan_hist(K - 1)

        def make_pass(hi_shift, lo_shift, width_mask, prefix):
            def p(cc, c2):
                bv = bits[pl.ds(cc * L, L)]
                m = lax.shift_right_logical(bv, hi_shift) == prefix
                bkt = jnp.bitwise_and(
                    lax.shift_right_logical(bv, lo_shift), width_mask
                )
                plsc.addupdate_scatter(hist, [lane_off + bkt], ones, mask=m)
                return c2

            return p

        lax.fori_loop(0, NCHUNK, make_pass(23, 15, 0xFF, b1), 0, unroll=8)
        b2, r0, _ = scan_hist(r0)
        p2 = b1 * 256 + b2
        lax.fori_loop(0, NCHUNK, make_pass(15, 7, 0xFF, p2), 0, unroll=8)
        b3, r0, _ = scan_hist(r0)
        p3 = p2 * 256 + b3
        lax.fori_loop(0, NCHUNK, make_pass(7, 0, 0x7F, p3), 0, unroll=8)
        b4, r0, hb = scan_hist(r0)
        t = p3 * 128 + b4
        need = r0 + 1

        def pass_e_fast(_):
            def body(cc, c2):
                bv = bits[pl.ds(cc * L, L)]
                mask[pl.ds(cc * L, L)] = jnp.where(bv <= t, 1, 0)
                return c2

            return lax.fori_loop(0, NCHUNK, body, 0, unroll=8)

        def pass_e_tie(_):
            def body(cc, cnt):
                bv = bits[pl.ds(cc * L, L)]
                lt = bv < t
                eq = bv == t
                eqi = jnp.where(eq, 1, 0)
                pc = plsc.cumsum(eqi) + cnt
                sel = jnp.logical_and(eq, pc <= need)
                mask[pl.ds(cc * L, L)] = jnp.where(
                    jnp.logical_or(lt, sel), 1, 0
                )
                return cnt + jnp.sum(eqi)

            return lax.fori_loop(0, NCHUNK, body, 0, unroll=8)

        lax.cond(hb == need, pass_e_fast, pass_e_tie, 0)
        pltpu.sync_copy(mask, out_hbm.at[row])
        return c

    lax.fori_loop(0, ROWS, row_body, 0)


def kernel(center):
    b, g, _ = center.shape
    idx_key = jax.random.key(42)
    rand_index = jax.random.randint(idx_key, (b,), 0, g)
    aidx = jnp.broadcast_to(
        (rand_index.astype(jnp.int32) * 3)[:, None], (b, L)
    )
    aidx = jnp.asarray(aidx, jnp.int32).reshape(b * L)
    flat = center.reshape(b, g * 3)
    out = _mask_kernel(flat, aidx)
    return out.astype(jnp.bool_)


# speedup vs baseline: 33.8747x; 1.6675x over previous
import functools

import jax
import jax.numpy as jnp
from jax import lax
from jax.experimental import pallas as pl
from jax.experimental.pallas import tpu as pltpu
from jax.experimental.pallas import tpu_sc as plsc

B = 256
G = 8192
K = int(0.6 * G)
NC, NS, L = 2, 16, 16
NW = NC * NS
ROWS = B // NW
NCHUNK = G // L
NBINS = 256
HIST = L * NBINS

_mesh = plsc.VectorSubcoreMesh(
    core_axis_name="c", subcore_axis_name="s", num_cores=NC, num_subcores=NS
)


@functools.partial(
    pl.kernel,
    out_type=jax.ShapeDtypeStruct((B, G), jnp.int32),
    mesh=_mesh,
    scratch_types=[
        pltpu.VMEM((G * 3,), jnp.float32),
        pltpu.VMEM((G,), jnp.int32),
        pltpu.VMEM((G,), jnp.int32),
        pltpu.VMEM((HIST,), jnp.int32),
        pltpu.VMEM((ROWS * L,), jnp.int32),
    ],
    compiler_params=pltpu.CompilerParams(
        use_tc_tiling_on_sc=False, needs_layout_passes=False
    ),
)
def _mask_kernel(center_hbm, aidx_hbm, out_hbm, pts, bits, mask, hist, ivv):
    wid = lax.axis_index("s") * NC + lax.axis_index("c")
    zeros = jnp.zeros((L,), jnp.int32)
    ones = jnp.ones((L,), jnp.int32)
    lane = lax.iota(jnp.int32, L)
    lane_off = lane * NBINS
    stride3 = lane * 3

    pltpu.sync_copy(aidx_hbm.at[pl.ds(wid * (ROWS * L), ROWS * L)], ivv)

    @plsc.parallel_loop(0, HIST // L, unroll=8)
    def _clr(i):
        hist[pl.ds(i * L, L)] = zeros

    def scan_hist(r0):
        def grp(j, carry):
            tot, b_acc, below, hb = carry
            s = zeros
            for l in range(L):
                off = j * L
                v = hist[pl.ds(off + l * NBINS, L)]
                hist[pl.ds(off + l * NBINS, L)] = zeros
                s = s + v
            cum = plsc.cumsum(s) + tot
            le = cum <= r0
            cross = jnp.logical_and(cum - s <= r0, jnp.logical_not(le))
            b_acc = b_acc + jnp.sum(jnp.where(le, 1, 0))
            below = below + jnp.sum(jnp.where(le, s, 0))
            hb = hb + jnp.sum(jnp.where(cross, s, 0))
            tot = tot + jnp.sum(s)
            return (tot, b_acc, below, hb)

        _, b, below, hb = lax.fori_loop(0, NBINS // L, grp, (0, 0, 0, 0))
        return b, r0 - below, hb

    def row_body(i, c):
        row = wid * ROWS + i
        pltpu.sync_copy(center_hbm.at[row], pts)
        iv = ivv[pl.ds(i * L, L)]
        ax = plsc.load_gather(pts, [iv])
        ay = plsc.load_gather(pts, [iv + 1])
        az = plsc.load_gather(pts, [iv + 2])

        @plsc.parallel_loop(0, NCHUNK, unroll=8)
        def pass_a(cc):
            i0 = cc * (3 * L) + stride3
            x = plsc.load_gather(pts, [i0])
            y = plsc.load_gather(pts, [i0 + 1])
            z = plsc.load_gather(pts, [i0 + 2])
            dx = x - ax
            dy = y - ay
            dz = z - az
            d2 = dx * dx + dy * dy + dz * dz
            bv = plsc.bitcast(d2, jnp.int32)
            bits[pl.ds(cc * L, L)] = bv
            bkt = lax.shift_right_logical(bv, 23)
            plsc.addupdate_scatter(hist, [lane_off + bkt], ones)

        b1, r0, _ = scan_hist(K - 1)

        def run_pass(hi_shift, lo_shift, width_mask, prefix):
            @plsc.parallel_loop(0, NCHUNK, unroll=8)
            def _p(cc):
                bv = bits[pl.ds(cc * L, L)]
                m = lax.shift_right_logical(bv, hi_shift) == prefix
                bkt = jnp.bitwise_and(
                    lax.shift_right_logical(bv, lo_shift), width_mask
                )
                plsc.addupdate_scatter(hist, [lane_off + bkt], ones, mask=m)

        run_pass(23, 15, 0xFF, b1)
        b2, r0, _ = scan_hist(r0)
        p2 = b1 * 256 + b2
        run_pass(15, 7, 0xFF, p2)
        b3, r0, _ = scan_hist(r0)
        p3 = p2 * 256 + b3
        run_pass(7, 0, 0x7F, p3)
        b4, r0, hb = scan_hist(r0)
        t = p3 * 128 + b4
        need = r0 + 1

        def pass_e_fast(_):
            @plsc.parallel_loop(0, NCHUNK, unroll=8)
            def body(cc):
                bv = bits[pl.ds(cc * L, L)]
                mask[pl.ds(cc * L, L)] = jnp.where(bv <= t, 1, 0)

            return 0

        def pass_e_tie(_):
            def body(cc, cnt):
                bv = bits[pl.ds(cc * L, L)]
                lt = bv < t
                eq = bv == t
                eqi = jnp.where(eq, 1, 0)
                pc = plsc.cumsum(eqi) + cnt
                sel = jnp.logical_and(eq, pc <= need)
                mask[pl.ds(cc * L, L)] = jnp.where(
                    jnp.logical_or(lt, sel), 1, 0
                )
                return cnt + jnp.sum(eqi)

            return lax.fori_loop(0, NCHUNK, body, 0, unroll=8)

        lax.cond(hb == need, pass_e_fast, pass_e_tie, 0)
        pltpu.sync_copy(mask, out_hbm.at[row])
        return c

    lax.fori_loop(0, ROWS, row_body, 0)


def kernel(center):
    b, g, _ = center.shape
    idx_key = jax.random.key(42)
    rand_index = jax.random.randint(idx_key, (b,), 0, g)
    aidx = jnp.broadcast_to(
        (rand_index.astype(jnp.int32) * 3)[:, None], (b, L)
    )
    aidx = jnp.asarray(aidx, jnp.int32).reshape(b * L)
    flat = center.reshape(b, g * 3)
    out = _mask_kernel(flat, aidx)
    return out.astype(jnp.bool_)


# speedup vs baseline: 36.3667x; 1.0736x over previous
import functools

import jax
import jax.numpy as jnp
from jax import lax
from jax.experimental import pallas as pl
from jax.experimental.pallas import tpu as pltpu
from jax.experimental.pallas import tpu_sc as plsc

B = 256
G = 8192
K = int(0.6 * G)
NC, NS, L = 2, 16, 16
NW = NC * NS
ROWS = B // NW
NCHUNK = G // L
NBINS = 256
HIST = L * NBINS

_mesh = plsc.VectorSubcoreMesh(
    core_axis_name="c", subcore_axis_name="s", num_cores=NC, num_subcores=NS
)


@functools.partial(
    pl.kernel,
    out_type=jax.ShapeDtypeStruct((B, G), jnp.int32),
    mesh=_mesh,
    scratch_types=[
        pltpu.VMEM((2 * G * 3,), jnp.float32),
        pltpu.VMEM((G,), jnp.int32),
        pltpu.VMEM((2 * G,), jnp.int32),
        pltpu.VMEM((HIST,), jnp.int32),
        pltpu.VMEM((ROWS * L,), jnp.int32),
        pltpu.SemaphoreType.DMA,
        pltpu.SemaphoreType.DMA,
        pltpu.SemaphoreType.DMA,
    ],
    compiler_params=pltpu.CompilerParams(
        use_tc_tiling_on_sc=False, needs_layout_passes=False
    ),
)
def _mask_kernel(
    center_hbm, aidx_hbm, out_hbm, pts, bits, mask, hist, ivv, isem, osem0, osem1
):
    wid = lax.axis_index("s") * NC + lax.axis_index("c")
    zeros = jnp.zeros((L,), jnp.int32)
    ones = jnp.ones((L,), jnp.int32)
    lane = lax.iota(jnp.int32, L)
    lane_off = lane * NBINS
    stride3 = lane * 3

    pltpu.sync_copy(aidx_hbm.at[pl.ds(wid * (ROWS * L), ROWS * L)], ivv)

    @plsc.parallel_loop(0, HIST // L, unroll=8)
    def _clr(i):
        hist[pl.ds(i * L, L)] = zeros

    def scan_hist(r0):
        def grp(j, carry):
            tot, b_acc, below, hb = carry
            s = zeros
            for l in range(L):
                off = j * L
                v = hist[pl.ds(off + l * NBINS, L)]
                hist[pl.ds(off + l * NBINS, L)] = zeros
                s = s + v
            cum = plsc.cumsum(s) + tot
            le = cum <= r0
            cross = jnp.logical_and(cum - s <= r0, jnp.logical_not(le))
            b_acc = b_acc + jnp.sum(jnp.where(le, 1, 0))
            below = below + jnp.sum(jnp.where(le, s, 0))
            hb = hb + jnp.sum(jnp.where(cross, s, 0))
            tot = tot + jnp.sum(s)
            return (tot, b_acc, below, hb)

        _, b, below, hb = lax.fori_loop(0, NBINS // L, grp, (0, 0, 0, 0))
        return b, r0 - below, hb

    pltpu.sync_copy(center_hbm.at[wid * ROWS], pts.at[pl.ds(0, G * 3)])

    def row_body(i, c):
        row = wid * ROWS + i
        par = jnp.bitwise_and(i, 1)
        base = par * (G * 3)
        nbase = (1 - par) * (G * 3)
        obase = par * G

        @pl.when(i < ROWS - 1)
        def _prefetch():
            pltpu.async_copy(
                center_hbm.at[row + 1], pts.at[pl.ds(nbase, G * 3)], isem
            )

        iv = ivv[pl.ds(i * L, L)] + base
        ax = plsc.load_gather(pts, [iv])
        ay = plsc.load_gather(pts, [iv + 1])
        az = plsc.load_gather(pts, [iv + 2])

        @plsc.parallel_loop(0, NCHUNK, unroll=8)
        def pass_a(cc):
            i0 = base + cc * (3 * L) + stride3
            x = plsc.load_gather(pts, [i0])
            y = plsc.load_gather(pts, [i0 + 1])
            z = plsc.load_gather(pts, [i0 + 2])
            dx = x - ax
            dy = y - ay
            dz = z - az
            d2 = dx * dx + dy * dy + dz * dz
            bv = plsc.bitcast(d2, jnp.int32)
            bits[pl.ds(cc * L, L)] = bv
            bkt = lax.shift_right_logical(bv, 23)
            plsc.addupdate_scatter(hist, [lane_off + bkt], ones)

        b1, r0, _ = scan_hist(K - 1)

        def run_pass(hi_shift, lo_shift, width_mask, prefix):
            @plsc.parallel_loop(0, NCHUNK, unroll=8)
            def _p(cc):
                bv = bits[pl.ds(cc * L, L)]
                m = lax.shift_right_logical(bv, hi_shift) == prefix
                bkt = jnp.bitwise_and(
                    lax.shift_right_logical(bv, lo_shift), width_mask
                )
                plsc.addupdate_scatter(hist, [lane_off + bkt], ones, mask=m)

        run_pass(23, 15, 0xFF, b1)
        b2, r0, _ = scan_hist(r0)
        p2 = b1 * 256 + b2
        run_pass(15, 7, 0xFF, p2)
        b3, r0, _ = scan_hist(r0)
        p3 = p2 * 256 + b3
        run_pass(7, 0, 0x7F, p3)
        b4, r0, hb = scan_hist(r0)
        t = p3 * 128 + b4
        need = r0 + 1

        @pl.when(jnp.logical_and(i >= 2, par == 0))
        def _drain0():
            pltpu.make_async_copy(
                mask.at[pl.ds(0, G)], out_hbm.at[row], osem0
            ).wait()

        @pl.when(jnp.logical_and(i >= 2, par == 1))
        def _drain1():
            pltpu.make_async_copy(
                mask.at[pl.ds(G, G)], out_hbm.at[row], osem1
            ).wait()

        def pass_e_fast(_):
            @plsc.parallel_loop(0, NCHUNK, unroll=8)
            def body(cc):
                bv = bits[pl.ds(cc * L, L)]
                mask[pl.ds(obase + cc * L, L)] = jnp.where(bv <= t, 1, 0)

            return 0

        def pass_e_tie(_):
            def body(cc, cnt):
                bv = bits[pl.ds(cc * L, L)]
                lt = bv < t
                eq = bv == t
                eqi = jnp.where(eq, 1, 0)
                pc = plsc.cumsum(eqi) + cnt
                sel = jnp.logical_and(eq, pc <= need)
                mask[pl.ds(obase + cc * L, L)] = jnp.where(
                    jnp.logical_or(lt, sel), 1, 0
                )
                return cnt + jnp.sum(eqi)

            return lax.fori_loop(0, NCHUNK, body, 0, unroll=8)

        lax.cond(hb == need, pass_e_fast, pass_e_tie, 0)

        @pl.when(par == 0)
        def _out0():
            pltpu.async_copy(mask.at[pl.ds(0, G)], out_hbm.at[row], osem0)

        @pl.when(par == 1)
        def _out1():
            pltpu.async_copy(mask.at[pl.ds(G, G)], out_hbm.at[row], osem1)

        @pl.when(i < ROWS - 1)
        def _wait_in():
            pltpu.make_async_copy(
                center_hbm.at[row + 1], pts.at[pl.ds(nbase, G * 3)], isem
            ).wait()

        return c

    lax.fori_loop(0, ROWS, row_body, 0)

    pltpu.make_async_copy(
        mask.at[pl.ds(0, G)], out_hbm.at[wid * ROWS + ROWS - 2], osem0
    ).wait()
    pltpu.make_async_copy(
        mask.at[pl.ds(G, G)], out_hbm.at[wid * ROWS + ROWS - 1], osem1
    ).wait()


def kernel(center):
    b, g, _ = center.shape
    idx_key = jax.random.key(42)
    rand_index = jax.random.randint(idx_key, (b,), 0, g)
    aidx = jnp.broadcast_to(
        (rand_index.astype(jnp.int32) * 3)[:, None], (b, L)
    )
    aidx = jnp.asarray(aidx, jnp.int32).reshape(b * L)
    flat = center.reshape(b, g * 3)
    out = _mask_kernel(flat, aidx)
    return out.astype(jnp.bool_)
